# baseline (speedup 1.0000x reference)
import functools
import math

import jax
import jax.numpy as jnp
from jax import lax
from jax.experimental import pallas as pl
from jax.experimental.pallas import tpu as pltpu
from jax.experimental.pallas import tpu_sc as plsc

B, T, D = 1, 2048, 768
H = 12
DH = D // H
E, K = 64, 2
HID = 1024
R, CH = 2, 4
CDH = D // CH
N = B * T
CAP = int(math.ceil(1.25 * N * K / E))
EPS = 1e-6
NBLK = 8
BLK = N // NBLK
BUF_ROWS = E * CAP
DUMP = BUF_ROWS
NW = 32
TPW = N // NW
CBLK = 512


def _rms(x, w):
    return x * w / jnp.sqrt(jnp.mean(x * x, axis=-1, keepdims=True) + EPS)


def _bf(v):
    return v.astype(jnp.bfloat16)


def _dot_t(a, b):
    return lax.dot_general(_bf(a), _bf(b), (((1,), (1,)), ((), ())),
                           preferred_element_type=jnp.float32)


def _dot_n(a, b):
    return lax.dot_general(_bf(a), _bf(b), (((1,), (0,)), ((), ())),
                           preferred_element_type=jnp.float32)


def _dot_t32(a, b):
    return lax.dot_general(a, b, (((1,), (1,)), ((), ())),
                           preferred_element_type=jnp.float32)


def _dot_n32(a, b):
    return lax.dot_general(a, b, (((1,), (0,)), ((), ())),
                           preferred_element_type=jnp.float32)


def _qkv_body(x_ref, w_ref, n1_ref, cos_ref, sin_ref, o_ref):
    h = _rms(x_ref[...], n1_ref[...])
    qkv = _dot_t32(h, w_ref[...])
    qk = qkv[:, : 2 * H * DH]
    lanes = qk.shape[1]
    ln = lax.broadcasted_iota(jnp.int32, (BLK, lanes), 1)
    even = (ln % 2) == 0
    rot = jnp.where(even, -jnp.roll(qk, -1, axis=1), jnp.roll(qk, 1, axis=1))
    cs = jnp.concatenate([cos_ref[...]] * (2 * H), axis=1)
    sn = jnp.concatenate([sin_ref[...]] * (2 * H), axis=1)
    o_ref[...] = jnp.concatenate([qk * cs + rot * sn, qkv[:, 2 * H * DH :]],
                                 axis=1)


def _qkv_call(x2d, wqkv, n1, cosf, sinf):
    return pl.pallas_call(
        _qkv_body,
        grid=(NBLK,),
        in_specs=[
            pl.BlockSpec((BLK, D), lambda i: (i, 0)),
            pl.BlockSpec(wqkv.shape, lambda i: (0, 0)),
            pl.BlockSpec((1, D), lambda i: (0, 0)),
            pl.BlockSpec((BLK, DH), lambda i: (i, 0)),
            pl.BlockSpec((BLK, DH), lambda i: (i, 0)),
        ],
        out_specs=pl.BlockSpec((BLK, 3 * D), lambda i: (i, 0)),
        out_shape=jax.ShapeDtypeStruct((N, 3 * D), jnp.float32),
    )(x2d, wqkv, n1, cosf, sinf)


def _attn_body(q_ref, k_ref, v_ref, o_ref, *, ibase, kcols):
    i = pl.program_id(1) + ibase
    row = lax.broadcasted_iota(jnp.int32, (BLK, kcols), 0) + i * BLK
    col = lax.broadcasted_iota(jnp.int32, (BLK, kcols), 1)
    causal = col <= row
    outs = []
    for sub in range(2):
        q = q_ref[:, sub * DH : (sub + 1) * DH] * (1.0 / math.sqrt(DH))
        k = k_ref[:, sub * DH : (sub + 1) * DH]
        v = v_ref[:, sub * DH : (sub + 1) * DH]
        p = jnp.exp(jnp.where(causal, _dot_t32(q, k), -1e30))
        l = jnp.sum(p, axis=-1, keepdims=True)
        outs.append(_dot_n32(p, v) / l)
    o_ref[...] = jnp.concatenate(outs, axis=1)


def _attn_part(qkv, ibase, nblk, kcols):
    return pl.pallas_call(
        functools.partial(_attn_body, ibase=ibase, kcols=kcols),
        grid=(H // 2, nblk),
        in_specs=[
            pl.BlockSpec((BLK, 2 * DH), lambda h, i: (i + ibase, h)),
            pl.BlockSpec((kcols, 2 * DH), lambda h, i: (0, H // 2 + h)),
            pl.BlockSpec((kcols, 2 * DH), lambda h, i: (0, H + h)),
        ],
        out_specs=pl.BlockSpec((BLK, 2 * DH), lambda h, i: (i, h)),
        out_shape=jax.ShapeDtypeStruct((nblk * BLK, D), jnp.float32),
    )(qkv, qkv, qkv)


def _attn_call(qkv):
    lo = _attn_part(qkv, 0, NBLK // 2, (NBLK // 2) * BLK)
    hi = _attn_part(qkv, NBLK // 2, NBLK // 2, N)
    return jnp.concatenate([lo, hi], axis=0)


def _post_attn_body(y_ref, x_ref, wo_ref, n2_ref, x1_ref, xn_ref):
    x1 = x_ref[...] + _dot_t32(y_ref[...], wo_ref[...])
    x1_ref[...] = x1
    xn_ref[...] = _rms(x1, n2_ref[...])


def _post_attn_call(y, x2d, wo, n2):
    return pl.pallas_call(
        _post_attn_body,
        grid=(NBLK,),
        in_specs=[
            pl.BlockSpec((BLK, D), lambda i: (i, 0)),
            pl.BlockSpec((BLK, D), lambda i: (i, 0)),
            pl.BlockSpec(wo.shape, lambda i: (0, 0)),
            pl.BlockSpec((1, D), lambda i: (0, 0)),
        ],
        out_specs=[
            pl.BlockSpec((BLK, D), lambda i: (i, 0)),
            pl.BlockSpec((BLK, D), lambda i: (i, 0)),
        ],
        out_shape=[
            jax.ShapeDtypeStruct((N, D), jnp.float32),
            jax.ShapeDtypeStruct((N, D), jnp.float32),
        ],
    )(y, x2d, wo, n2)


def _router_body(xn_ref, gw_ref, tp0_ref, tp1_ref, gm0_ref, gm1_ref,
                 sw0_ref, sw1_ref, sg0_ref, sg1_ref):
    logits = lax.dot_general(xn_ref[...], gw_ref[...], (((1,), (1,)), ((), ())),
                             preferred_element_type=jnp.float32)
    mx = jnp.max(logits, axis=-1, keepdims=True)
    ex = jnp.exp(logits - mx)
    probs = ex / jnp.sum(ex, axis=-1, keepdims=True)

    eid = lax.broadcasted_iota(jnp.int32, (N, E), 1)
    tp0 = jnp.max(probs, axis=-1, keepdims=True)
    ti0 = jnp.min(jnp.where(probs == tp0, eid, E), axis=-1, keepdims=True)
    oh0 = (eid == ti0)
    p2 = jnp.where(oh0, -1.0, probs)
    tp1 = jnp.max(p2, axis=-1, keepdims=True)
    ti1 = jnp.min(jnp.where(p2 == tp1, eid, E), axis=-1, keepdims=True)
    oh1 = (eid == ti1)
    den = tp0 + tp1
    tp0_ref[...] = tp0 / den
    tp1_ref[...] = tp1 / den

    cmb = jnp.where(oh0 | oh1, 1.0, 0.0)
    rr = lax.broadcasted_iota(jnp.int32, (BLK, BLK), 0)
    cc = lax.broadcasted_iota(jnp.int32, (BLK, BLK), 1)
    ltri = jnp.where(cc < rr, 1.0, 0.0)
    parts = []
    carry = jnp.zeros((1, E), jnp.float32)
    for b in range(NBLK):
        blk = cmb[b * BLK : (b + 1) * BLK]
        parts.append(_dot_n(ltri, blk) + carry)
        carry = carry + jnp.sum(blk, axis=0, keepdims=True)
    cum = jnp.concatenate(parts, axis=0)

    pos0 = jnp.sum(jnp.where(oh0, cum, 0.0), axis=-1, keepdims=True)
    pos1 = jnp.sum(jnp.where(oh1, cum, 0.0), axis=-1, keepdims=True)
    pos0 = pos0.astype(jnp.int32)
    pos1 = pos1.astype(jnp.int32)
    keep0 = pos0 < CAP
    keep1 = pos1 < CAP
    gm0_ref[...] = jnp.where(keep0, 1.0, 0.0)
    gm1_ref[...] = jnp.where(keep1, 1.0, 0.0)
    sw0_ref[...] = jnp.where(keep0, ti0 * CAP + pos0, DUMP)
    sw1_ref[...] = jnp.where(keep1, ti1 * CAP + pos1, DUMP)
    sg0_ref[...] = ti0 * CAP + jnp.where(keep0, pos0, 0)
    sg1_ref[...] = ti1 * CAP + jnp.where(keep1, pos1, 0)


def _router_call(xn2, gate_w):
    o32 = lambda s: jax.ShapeDtypeStruct(s, jnp.int32)
    of = lambda s: jax.ShapeDtypeStruct(s, jnp.float32)
    return pl.pallas_call(
        _router_body,
        out_shape=[of((N, 1)), of((N, 1)), of((N, 1)), of((N, 1)),
                   o32((N, 1)), o32((N, 1)), o32((N, 1)), o32((N, 1))],
    )(xn2, gate_w)


_SC_MESH = dict(core_axis_name="c", subcore_axis_name="s")


def _sc_wid():
    return lax.axis_index("s") * 2 + lax.axis_index("c")


def _sc_dispatch_body(xn_hbm, sw0_hbm, sw1_hbm, buf_hbm,
                      idx0_v, idx1_v, rows_v, sem):
    base = _sc_wid() * TPW
    pltpu.sync_copy(xn_hbm.at[pl.ds(base, TPW)], rows_v)
    pltpu.sync_copy(sw0_hbm.at[pl.ds(base, TPW)], idx0_v)
    pltpu.sync_copy(sw1_hbm.at[pl.ds(base, TPW)], idx1_v)
    pltpu.async_copy(rows_v, buf_hbm.at[idx0_v], sem).wait()
    pltpu.async_copy(rows_v, buf_hbm.at[idx1_v], sem).wait()


def _sc_dispatch_call(xn2, sw0, sw1):
    f = pl.kernel(
        _sc_dispatch_body,
        out_type=jax.ShapeDtypeStruct((BUF_ROWS + 8, D), jnp.float32),
        mesh=plsc.VectorSubcoreMesh(**_SC_MESH),
        scratch_types=[
            pltpu.VMEM((TPW,), jnp.int32),
            pltpu.VMEM((TPW,), jnp.int32),
            pltpu.VMEM((TPW, D), jnp.float32),
            pltpu.SemaphoreType.DMA,
        ],
    )
    return f(xn2, sw0, sw1)


def _sc_gather_body(ob_hbm, sg0_hbm, sg1_hbm, y0_hbm, y1_hbm,
                    idx_v, rows_v, sem):
    base = _sc_wid() * TPW
    pltpu.sync_copy(sg0_hbm.at[pl.ds(base, TPW)], idx_v)
    pltpu.async_copy(ob_hbm.at[idx_v], rows_v, sem).wait()
    pltpu.sync_copy(rows_v, y0_hbm.at[pl.ds(base, TPW)])
    pltpu.sync_copy(sg1_hbm.at[pl.ds(base, TPW)], idx_v)
    pltpu.async_copy(ob_hbm.at[idx_v], rows_v, sem).wait()
    pltpu.sync_copy(rows_v, y1_hbm.at[pl.ds(base, TPW)])


def _sc_gather_call(outbuf, sg0, sg1):
    f = pl.kernel(
        _sc_gather_body,
        out_type=(jax.ShapeDtypeStruct((N, D), jnp.float32),
                  jax.ShapeDtypeStruct((N, D), jnp.float32)),
        mesh=plsc.VectorSubcoreMesh(**_SC_MESH),
        scratch_types=[
            pltpu.VMEM((TPW,), jnp.int32),
            pltpu.VMEM((TPW, D), jnp.float32),
            pltpu.SemaphoreType.DMA,
        ],
    )
    return f(outbuf, sg0, sg1)


def _expert_body(xb_ref, w13_ref, w2_ref, o_ref):
    xb = xb_ref[...]
    z = 2.0 * xb
    gu = _dot_t(z, w13_ref[0])
    g = gu[:, :HID]
    u = gu[:, HID:]
    act = g * jax.nn.sigmoid(g) * u
    f = _dot_t(act, w2_ref[0])
    o_ref[...] = xb + f


def _expert_call(buf, w13s, w2s):
    return pl.pallas_call(
        _expert_body,
        grid=(E,),
        in_specs=[
            pl.BlockSpec((CAP, D), lambda e: (e, 0)),
            pl.BlockSpec((1, 2 * HID, D), lambda e: (e, 0, 0)),
            pl.BlockSpec((1, D, HID), lambda e: (e, 0, 0)),
        ],
        out_specs=pl.BlockSpec((CAP, D), lambda e: (e, 0)),
        out_shape=jax.ShapeDtypeStruct((BUF_ROWS, D), jnp.float32),
    )(buf, w13s, w2s)


def _gelu_exact(v):
    return 0.5 * v * (1.0 + lax.erf(v * (1.0 / math.sqrt(2.0))))


def _collab_body(y0_ref, y1_ref, gm0_ref, gm1_ref, tp0_ref, tp1_ref, x1_ref,
                 med_ref, ipw_ref, ipb_ref, opw_ref, opb_ref, cn1_ref,
                 cn2_ref, cw1_ref, cw2_ref, fw_ref, fb_ref, wom_ref, o_ref):
    seq = jnp.concatenate([
        jnp.broadcast_to(med_ref[...], (CBLK, D)),
        y0_ref[...] * gm0_ref[...],
        y1_ref[...] * gm1_ref[...],
    ], axis=0)

    dsel = lax.broadcasted_iota(jnp.int32, (D, CH), 0) // CDH
    hsel = lax.broadcasted_iota(jnp.int32, (D, CH), 1)
    red = jnp.where(dsel == hsel, 1.0, 0.0)
    exp_sel = red.T

    cn1 = cn1_ref[...]
    cn2 = cn2_ref[...]
    ipw = ipw_ref[...]
    ipb = ipb_ref[...]
    opw = opw_ref[...]
    opb = opb_ref[...]
    cw1 = cw1_ref[...]
    cw2 = cw2_ref[...]

    for _ in range(R):
        stk = _rms(seq, cn1)
        qkv = _dot_t(stk, ipw) + ipb
        q = [qkv[i * CBLK : (i + 1) * CBLK, :D] for i in range(3)]
        k = [qkv[i * CBLK : (i + 1) * CBLK, D : 2 * D] for i in range(3)]
        v = [qkv[i * CBLK : (i + 1) * CBLK, 2 * D :] for i in range(3)]
        outs = []
        for i in range(3):
            sc = [_dot_n(q[i] * k[j], red) * (1.0 / math.sqrt(CDH))
                  for j in range(3)]
            mmax = jnp.maximum(jnp.maximum(sc[0], sc[1]), sc[2])
            es = [jnp.exp(s_ - mmax) for s_ in sc]
            den = es[0] + es[1] + es[2]
            acc = jnp.zeros((CBLK, D), jnp.float32)
            for j in range(3):
                a = _dot_n(es[j] / den, exp_sel)
                acc = acc + a * v[j]
            outs.append(acc)
        proj = _dot_t(jnp.concatenate(outs, axis=0), opw) + opb
        seq = seq + proj
        f1 = _gelu_exact(_dot_t(_rms(seq, cn2), cw1))
        seq = seq + _dot_t(f1, cw2)

    m = seq[:CBLK]
    s0 = seq[CBLK : 2 * CBLK]
    s1 = seq[2 * CBLK :]
    agg = tp0_ref[...] * s0 + tp1_ref[...] * s1
    gate = jax.nn.sigmoid(
        jnp.sum(m * fw_ref[...], axis=-1, keepdims=True) + fb_ref[...])
    fused = gate * m + (1.0 - gate) * agg
    o_ref[...] = x1_ref[...] + _dot_t(fused, wom_ref[...])


def _collab_call(y0, y1, gm0, gm1, tp0, tp1, x1, med, ipw, ipb, opw, opb,
                 cn1, cn2, cw1, cw2, fw, fb, wom):
    blk = lambda s: pl.BlockSpec(s, lambda i: (i, 0))
    full = lambda a: pl.BlockSpec(a.shape, lambda i: tuple(0 for _ in a.shape))
    return pl.pallas_call(
        _collab_body,
        grid=(N // CBLK,),
        in_specs=[
            blk((CBLK, D)), blk((CBLK, D)),
            blk((CBLK, 1)), blk((CBLK, 1)), blk((CBLK, 1)), blk((CBLK, 1)),
            blk((CBLK, D)),
            full(med), full(ipw), full(ipb), full(opw), full(opb),
            full(cn1), full(cn2), full(cw1), full(cw2), full(fw), full(fb),
            full(wom),
        ],
        out_specs=pl.BlockSpec((CBLK, D), lambda i: (i, 0)),
        out_shape=jax.ShapeDtypeStruct((N, D), jnp.float32),
    )(y0, y1, gm0, gm1, tp0, tp1, x1, med, ipw, ipb, opw, opb, cn1, cn2,
      cw1, cw2, fw, fb, wom)


def kernel(x, freqs_cis, norm1_w, norm2_w, wqkv, wo_attn, gate_w, w13s, w2s,
           mediator, in_proj_w, in_proj_b, out_proj_w, out_proj_b, collab_n1,
           collab_n2, collab_w1, collab_w2, fuse_w, fuse_b, wo_moe):
    x2d = x.reshape(N, D)
    cosf = jnp.repeat(freqs_cis[..., 0], 2, axis=1)
    sinf = jnp.repeat(freqs_cis[..., 1], 2, axis=1)

    qkv = _qkv_call(x2d, wqkv, norm1_w.reshape(1, D), cosf, sinf)
    y = _attn_call(qkv)
    x1, xn2 = _post_attn_call(y, x2d, wo_attn, norm2_w.reshape(1, D))

    tp0, tp1, gm0, gm1, sw0, sw1, sg0, sg1 = _router_call(xn2, gate_w)

    buf = _sc_dispatch_call(xn2, sw0.reshape(N), sw1.reshape(N))
    outbuf = _expert_call(buf, w13s, w2s)
    y0, y1 = _sc_gather_call(outbuf, sg0.reshape(N), sg1.reshape(N))

    out = _collab_call(
        y0, y1, gm0, gm1, tp0, tp1, x1,
        mediator.reshape(1, D),
        in_proj_w, in_proj_b.reshape(1, 3 * D),
        out_proj_w, out_proj_b.reshape(1, D),
        collab_n1.reshape(1, D), collab_n2.reshape(1, D),
        collab_w1, collab_w2,
        fuse_w.reshape(1, D), fuse_b.reshape(1, 1), wo_moe)
    return out.reshape(B, T, D)


# speedup vs baseline: 1.0279x; 1.0279x over previous
import functools
import math

import jax
import jax.numpy as jnp
from jax import lax
from jax.experimental import pallas as pl
from jax.experimental.pallas import tpu as pltpu
from jax.experimental.pallas import tpu_sc as plsc

B, T, D = 1, 2048, 768
H = 12
DH = D // H
E, K = 64, 2
HID = 1024
R, CH = 2, 4
CDH = D // CH
N = B * T
CAP = int(math.ceil(1.25 * N * K / E))
EPS = 1e-6
NBLK = 8
BLK = N // NBLK
BUF_ROWS = E * CAP
DUMP = BUF_ROWS
NW = 32
TPW = N // NW
CBLK = 256


def _rms(x, w):
    return x * w / jnp.sqrt(jnp.mean(x * x, axis=-1, keepdims=True) + EPS)


def _bf(v):
    return v.astype(jnp.bfloat16)


def _dot_t(a, b):
    return lax.dot_general(_bf(a), _bf(b), (((1,), (1,)), ((), ())),
                           preferred_element_type=jnp.float32)


def _dot_n(a, b):
    return lax.dot_general(_bf(a), _bf(b), (((1,), (0,)), ((), ())),
                           preferred_element_type=jnp.float32)


def _dot_t32(a, b):
    return lax.dot_general(a, b, (((1,), (1,)), ((), ())),
                           preferred_element_type=jnp.float32)


def _dot_n32(a, b):
    return lax.dot_general(a, b, (((1,), (0,)), ((), ())),
                           preferred_element_type=jnp.float32)


def _qkv_body(x_ref, w_ref, n1_ref, cos_ref, sin_ref, o_ref):
    h = _rms(x_ref[...], n1_ref[...])
    qkv = _dot_t32(h, w_ref[...])
    qk = qkv[:, : 2 * H * DH]
    lanes = qk.shape[1]
    ln = lax.broadcasted_iota(jnp.int32, (BLK, lanes), 1)
    even = (ln % 2) == 0
    rot = jnp.where(even, -jnp.roll(qk, -1, axis=1), jnp.roll(qk, 1, axis=1))
    cs = jnp.concatenate([cos_ref[...]] * (2 * H), axis=1)
    sn = jnp.concatenate([sin_ref[...]] * (2 * H), axis=1)
    o_ref[...] = jnp.concatenate([qk * cs + rot * sn, qkv[:, 2 * H * DH :]],
                                 axis=1)


def _qkv_call(x2d, wqkv, n1, cosf, sinf):
    return pl.pallas_call(
        _qkv_body,
        grid=(NBLK,),
        in_specs=[
            pl.BlockSpec((BLK, D), lambda i: (i, 0)),
            pl.BlockSpec(wqkv.shape, lambda i: (0, 0)),
            pl.BlockSpec((1, D), lambda i: (0, 0)),
            pl.BlockSpec((BLK, DH), lambda i: (i, 0)),
            pl.BlockSpec((BLK, DH), lambda i: (i, 0)),
        ],
        out_specs=pl.BlockSpec((BLK, 3 * D), lambda i: (i, 0)),
        out_shape=jax.ShapeDtypeStruct((N, 3 * D), jnp.float32),
    )(x2d, wqkv, n1, cosf, sinf)


def _attn_body(q_ref, k_ref, v_ref, o_ref, *, ibase, kcols):
    i = pl.program_id(1) + ibase
    row = lax.broadcasted_iota(jnp.int32, (BLK, kcols), 0) + i * BLK
    col = lax.broadcasted_iota(jnp.int32, (BLK, kcols), 1)
    causal = col <= row
    outs = []
    for sub in range(2):
        q = q_ref[:, sub * DH : (sub + 1) * DH] * (1.0 / math.sqrt(DH))
        k = k_ref[:, sub * DH : (sub + 1) * DH]
        v = v_ref[:, sub * DH : (sub + 1) * DH]
        p = jnp.exp(jnp.where(causal, _dot_t32(q, k), -1e30))
        l = jnp.sum(p, axis=-1, keepdims=True)
        outs.append(_dot_n32(p, v) / l)
    o_ref[...] = jnp.concatenate(outs, axis=1)


def _attn_part(qkv, ibase, nblk, kcols):
    return pl.pallas_call(
        functools.partial(_attn_body, ibase=ibase, kcols=kcols),
        grid=(H // 2, nblk),
        in_specs=[
            pl.BlockSpec((BLK, 2 * DH), lambda h, i: (i + ibase, h)),
            pl.BlockSpec((kcols, 2 * DH), lambda h, i: (0, H // 2 + h)),
            pl.BlockSpec((kcols, 2 * DH), lambda h, i: (0, H + h)),
        ],
        out_specs=pl.BlockSpec((BLK, 2 * DH), lambda h, i: (i, h)),
        out_shape=jax.ShapeDtypeStruct((nblk * BLK, D), jnp.float32),
    )(qkv, qkv, qkv)


def _attn_call(qkv):
    lo = _attn_part(qkv, 0, NBLK // 2, (NBLK // 2) * BLK)
    hi = _attn_part(qkv, NBLK // 2, NBLK // 2, N)
    return jnp.concatenate([lo, hi], axis=0)


def _post_attn_body(y_ref, x_ref, wo_ref, n2_ref, x1_ref, xn_ref):
    x1 = x_ref[...] + _dot_t32(y_ref[...], wo_ref[...])
    x1_ref[...] = x1
    xn_ref[...] = _rms(x1, n2_ref[...])


def _post_attn_call(y, x2d, wo, n2):
    return pl.pallas_call(
        _post_attn_body,
        grid=(NBLK,),
        in_specs=[
            pl.BlockSpec((BLK, D), lambda i: (i, 0)),
            pl.BlockSpec((BLK, D), lambda i: (i, 0)),
            pl.BlockSpec(wo.shape, lambda i: (0, 0)),
            pl.BlockSpec((1, D), lambda i: (0, 0)),
        ],
        out_specs=[
            pl.BlockSpec((BLK, D), lambda i: (i, 0)),
            pl.BlockSpec((BLK, D), lambda i: (i, 0)),
        ],
        out_shape=[
            jax.ShapeDtypeStruct((N, D), jnp.float32),
            jax.ShapeDtypeStruct((N, D), jnp.float32),
        ],
    )(y, x2d, wo, n2)


def _router_body(xn_ref, gw_ref, tp0_ref, tp1_ref, gm0_ref, gm1_ref,
                 sw0_ref, sw1_ref, sg0_ref, sg1_ref):
    logits = lax.dot_general(xn_ref[...], gw_ref[...], (((1,), (1,)), ((), ())),
                             preferred_element_type=jnp.float32)
    mx = jnp.max(logits, axis=-1, keepdims=True)
    ex = jnp.exp(logits - mx)
    probs = ex / jnp.sum(ex, axis=-1, keepdims=True)

    eid = lax.broadcasted_iota(jnp.int32, (N, E), 1)
    tp0 = jnp.max(probs, axis=-1, keepdims=True)
    ti0 = jnp.min(jnp.where(probs == tp0, eid, E), axis=-1, keepdims=True)
    oh0 = (eid == ti0)
    p2 = jnp.where(oh0, -1.0, probs)
    tp1 = jnp.max(p2, axis=-1, keepdims=True)
    ti1 = jnp.min(jnp.where(p2 == tp1, eid, E), axis=-1, keepdims=True)
    oh1 = (eid == ti1)
    den = tp0 + tp1
    tp0_ref[...] = tp0 / den
    tp1_ref[...] = tp1 / den

    cmb = jnp.where(oh0 | oh1, 1.0, 0.0)
    rr = lax.broadcasted_iota(jnp.int32, (BLK, BLK), 0)
    cc = lax.broadcasted_iota(jnp.int32, (BLK, BLK), 1)
    ltri = jnp.where(cc < rr, 1.0, 0.0)
    parts = []
    carry = jnp.zeros((1, E), jnp.float32)
    for b in range(NBLK):
        blk = cmb[b * BLK : (b + 1) * BLK]
        parts.append(_dot_n(ltri, blk) + carry)
        carry = carry + jnp.sum(blk, axis=0, keepdims=True)
    cum = jnp.concatenate(parts, axis=0)

    pos0 = jnp.sum(jnp.where(oh0, cum, 0.0), axis=-1, keepdims=True)
    pos1 = jnp.sum(jnp.where(oh1, cum, 0.0), axis=-1, keepdims=True)
    pos0 = pos0.astype(jnp.int32)
    pos1 = pos1.astype(jnp.int32)
    keep0 = pos0 < CAP
    keep1 = pos1 < CAP
    gm0_ref[...] = jnp.where(keep0, 1.0, 0.0)
    gm1_ref[...] = jnp.where(keep1, 1.0, 0.0)
    sw0_ref[...] = jnp.where(keep0, ti0 * CAP + pos0, DUMP)
    sw1_ref[...] = jnp.where(keep1, ti1 * CAP + pos1, DUMP)
    sg0_ref[...] = ti0 * CAP + jnp.where(keep0, pos0, 0)
    sg1_ref[...] = ti1 * CAP + jnp.where(keep1, pos1, 0)


def _router_call(xn2, gate_w):
    o32 = lambda s: jax.ShapeDtypeStruct(s, jnp.int32)
    of = lambda s: jax.ShapeDtypeStruct(s, jnp.float32)
    return pl.pallas_call(
        _router_body,
        out_shape=[of((N, 1)), of((N, 1)), of((N, 1)), of((N, 1)),
                   o32((N, 1)), o32((N, 1)), o32((N, 1)), o32((N, 1))],
    )(xn2, gate_w)


_SC_MESH = dict(core_axis_name="c", subcore_axis_name="s")


def _sc_wid():
    return lax.axis_index("s") * 2 + lax.axis_index("c")


def _sc_dispatch_body(xn_hbm, sw0_hbm, sw1_hbm, buf_hbm,
                      idx0_v, idx1_v, rows_v, sem):
    base = _sc_wid() * TPW
    pltpu.sync_copy(xn_hbm.at[pl.ds(base, TPW)], rows_v)
    pltpu.sync_copy(sw0_hbm.at[pl.ds(base, TPW)], idx0_v)
    pltpu.sync_copy(sw1_hbm.at[pl.ds(base, TPW)], idx1_v)
    pltpu.async_copy(rows_v, buf_hbm.at[idx0_v], sem).wait()
    pltpu.async_copy(rows_v, buf_hbm.at[idx1_v], sem).wait()


def _sc_dispatch_call(xn2, sw0, sw1):
    f = pl.kernel(
        _sc_dispatch_body,
        out_type=jax.ShapeDtypeStruct((BUF_ROWS + 8, D), jnp.float32),
        mesh=plsc.VectorSubcoreMesh(**_SC_MESH),
        scratch_types=[
            pltpu.VMEM((TPW,), jnp.int32),
            pltpu.VMEM((TPW,), jnp.int32),
            pltpu.VMEM((TPW, D), jnp.float32),
            pltpu.SemaphoreType.DMA,
        ],
    )
    return f(xn2, sw0, sw1)


def _sc_gather_body(ob_hbm, sg0_hbm, sg1_hbm, y0_hbm, y1_hbm,
                    idx_v, rows_v, sem):
    base = _sc_wid() * TPW
    pltpu.sync_copy(sg0_hbm.at[pl.ds(base, TPW)], idx_v)
    pltpu.async_copy(ob_hbm.at[idx_v], rows_v, sem).wait()
    pltpu.sync_copy(rows_v, y0_hbm.at[pl.ds(base, TPW)])
    pltpu.sync_copy(sg1_hbm.at[pl.ds(base, TPW)], idx_v)
    pltpu.async_copy(ob_hbm.at[idx_v], rows_v, sem).wait()
    pltpu.sync_copy(rows_v, y1_hbm.at[pl.ds(base, TPW)])


def _sc_gather_call(outbuf, sg0, sg1):
    f = pl.kernel(
        _sc_gather_body,
        out_type=(jax.ShapeDtypeStruct((N, D), jnp.float32),
                  jax.ShapeDtypeStruct((N, D), jnp.float32)),
        mesh=plsc.VectorSubcoreMesh(**_SC_MESH),
        scratch_types=[
            pltpu.VMEM((TPW,), jnp.int32),
            pltpu.VMEM((TPW, D), jnp.float32),
            pltpu.SemaphoreType.DMA,
        ],
    )
    return f(outbuf, sg0, sg1)


def _expert_body(xb_ref, w13_ref, w2_ref, o_ref):
    xb = xb_ref[...]
    z = 2.0 * xb
    gu = _dot_t(z, w13_ref[0])
    g = gu[:, :HID]
    u = gu[:, HID:]
    act = g * jax.nn.sigmoid(g) * u
    f = _dot_t(act, w2_ref[0])
    o_ref[...] = xb + f


def _expert_call(buf, w13s, w2s):
    return pl.pallas_call(
        _expert_body,
        grid=(E,),
        in_specs=[
            pl.BlockSpec((CAP, D), lambda e: (e, 0)),
            pl.BlockSpec((1, 2 * HID, D), lambda e: (e, 0, 0)),
            pl.BlockSpec((1, D, HID), lambda e: (e, 0, 0)),
        ],
        out_specs=pl.BlockSpec((CAP, D), lambda e: (e, 0)),
        out_shape=jax.ShapeDtypeStruct((BUF_ROWS, D), jnp.float32),
    )(buf, w13s, w2s)


def _gelu_exact(v):
    return 0.5 * v * (1.0 + lax.erf(v * (1.0 / math.sqrt(2.0))))


def _collab_body(y0_ref, y1_ref, gm0_ref, gm1_ref, tp0_ref, tp1_ref, x1_ref,
                 med_ref, ipw_ref, ipb_ref, opw_ref, opb_ref, cn1_ref,
                 cn2_ref, cw1_ref, cw2_ref, fw_ref, fb_ref, wom_ref, o_ref):
    seq = jnp.concatenate([
        jnp.broadcast_to(med_ref[...], (CBLK, D)),
        y0_ref[...] * gm0_ref[...],
        y1_ref[...] * gm1_ref[...],
    ], axis=0)

    dsel = lax.broadcasted_iota(jnp.int32, (D, CH), 0) // CDH
    hsel = lax.broadcasted_iota(jnp.int32, (D, CH), 1)
    red = jnp.where(dsel == hsel, 1.0, 0.0)
    exp_sel = red.T

    cn1 = cn1_ref[...]
    cn2 = cn2_ref[...]
    ipw = ipw_ref[...]
    ipb = ipb_ref[...]
    opw = opw_ref[...]
    opb = opb_ref[...]
    cw1 = cw1_ref[...]
    cw2 = cw2_ref[...]

    for _ in range(R):
        stk = _rms(seq, cn1)
        qkv = _dot_t(stk, ipw) + ipb
        q = [qkv[i * CBLK : (i + 1) * CBLK, :D] for i in range(3)]
        k = [qkv[i * CBLK : (i + 1) * CBLK, D : 2 * D] for i in range(3)]
        v = [qkv[i * CBLK : (i + 1) * CBLK, 2 * D :] for i in range(3)]
        outs = []
        for i in range(3):
            sc = [_dot_n(q[i] * k[j], red) * (1.0 / math.sqrt(CDH))
                  for j in range(3)]
            mmax = jnp.maximum(jnp.maximum(sc[0], sc[1]), sc[2])
            es = [jnp.exp(s_ - mmax) for s_ in sc]
            den = es[0] + es[1] + es[2]
            acc = jnp.zeros((CBLK, D), jnp.float32)
            for j in range(3):
                a = _dot_n(es[j] / den, exp_sel)
                acc = acc + a * v[j]
            outs.append(acc)
        proj = _dot_t(jnp.concatenate(outs, axis=0), opw) + opb
        seq = seq + proj
        f1 = _gelu_exact(_dot_t(_rms(seq, cn2), cw1))
        seq = seq + _dot_t(f1, cw2)

    m = seq[:CBLK]
    s0 = seq[CBLK : 2 * CBLK]
    s1 = seq[2 * CBLK :]
    agg = tp0_ref[...] * s0 + tp1_ref[...] * s1
    gate = jax.nn.sigmoid(
        jnp.sum(m * fw_ref[...], axis=-1, keepdims=True) + fb_ref[...])
    fused = gate * m + (1.0 - gate) * agg
    o_ref[...] = x1_ref[...] + _dot_t(fused, wom_ref[...])


def _collab_call(y0, y1, gm0, gm1, tp0, tp1, x1, med, ipw, ipb, opw, opb,
                 cn1, cn2, cw1, cw2, fw, fb, wom):
    blk = lambda s: pl.BlockSpec(s, lambda i: (i, 0))
    full = lambda a: pl.BlockSpec(a.shape, lambda i: tuple(0 for _ in a.shape))
    return pl.pallas_call(
        _collab_body,
        grid=(N // CBLK,),
        in_specs=[
            blk((CBLK, D)), blk((CBLK, D)),
            blk((CBLK, 1)), blk((CBLK, 1)), blk((CBLK, 1)), blk((CBLK, 1)),
            blk((CBLK, D)),
            full(med), full(ipw), full(ipb), full(opw), full(opb),
            full(cn1), full(cn2), full(cw1), full(cw2), full(fw), full(fb),
            full(wom),
        ],
        out_specs=pl.BlockSpec((CBLK, D), lambda i: (i, 0)),
        out_shape=jax.ShapeDtypeStruct((N, D), jnp.float32),
    )(y0, y1, gm0, gm1, tp0, tp1, x1, med, ipw, ipb, opw, opb, cn1, cn2,
      cw1, cw2, fw, fb, wom)


def kernel(x, freqs_cis, norm1_w, norm2_w, wqkv, wo_attn, gate_w, w13s, w2s,
           mediator, in_proj_w, in_proj_b, out_proj_w, out_proj_b, collab_n1,
           collab_n2, collab_w1, collab_w2, fuse_w, fuse_b, wo_moe):
    x2d = x.reshape(N, D)
    cosf = jnp.repeat(freqs_cis[..., 0], 2, axis=1)
    sinf = jnp.repeat(freqs_cis[..., 1], 2, axis=1)

    qkv = _qkv_call(x2d, wqkv, norm1_w.reshape(1, D), cosf, sinf)
    y = _attn_call(qkv)
    x1, xn2 = _post_attn_call(y, x2d, wo_attn, norm2_w.reshape(1, D))

    tp0, tp1, gm0, gm1, sw0, sw1, sg0, sg1 = _router_call(xn2, gate_w)

    buf = _sc_dispatch_call(xn2, sw0.reshape(N), sw1.reshape(N))
    outbuf = _expert_call(buf, w13s, w2s)
    y0, y1 = _sc_gather_call(outbuf, sg0.reshape(N), sg1.reshape(N))

    out = _collab_call(
        y0, y1, gm0, gm1, tp0, tp1, x1,
        mediator.reshape(1, D),
        in_proj_w, in_proj_b.reshape(1, 3 * D),
        out_proj_w, out_proj_b.reshape(1, D),
        collab_n1.reshape(1, D), collab_n2.reshape(1, D),
        collab_w1, collab_w2,
        fuse_w.reshape(1, D), fuse_b.reshape(1, 1), wo_moe)
    return out.reshape(B, T, D)
